# Initial kernel scaffold; baseline (speedup 1.0000x reference)
import jax
import jax.numpy as jnp
from jax.experimental import pallas as pl


def kernel(x, edge_index, edge_attr, We1, be1, We2, be2, Wp1, bp1, Wp2, bp2, Wg1, bg1, Wg2, bg2, Wb1, bb1, Wb2, bb2):
    raise NotImplementedError("write your pallas kernel here")


# speedup vs baseline: 1.1319x; 1.1319x over previous
import functools

import jax
import jax.numpy as jnp
from jax import lax
from jax.experimental import pallas as pl
from jax.experimental.pallas import tpu as pltpu
from jax.experimental.pallas import tpu_sc as plsc

NC = 2
NS = 16
NW = NC * NS
CH = 80
CHS = 512

_GDN = lax.GatherDimensionNumbers(
    offset_dims=(), collapsed_slice_dims=(0,), start_index_map=(0,))


def _splat(vec, q):
    idx = jnp.full((16, 1), q, jnp.int32)
    return lax.gather(vec, idx, dimension_numbers=_GDN, slice_sizes=(1,),
                      mode=lax.GatherScatterMode.PROMISE_IN_BOUNDS)


@functools.cache
def _gather_fn(n, d, e):
    epw = e // NW
    cpw = epw // CH
    mesh = plsc.VectorSubcoreMesh(core_axis_name="c", subcore_axis_name="s")

    @functools.partial(
        pl.kernel, mesh=mesh,
        out_type=[jax.ShapeDtypeStruct((e, d), jnp.float32),
                  jax.ShapeDtypeStruct((e, d), jnp.float32)],
        scratch_types=[
            pltpu.VMEM((cpw, CH), jnp.int32),
            pltpu.VMEM((cpw, CH), jnp.int32),
            pltpu.VMEM((CH, d), jnp.float32),
            pltpu.VMEM((CH, d), jnp.float32),
            pltpu.SemaphoreType.DMA,
            pltpu.SemaphoreType.DMA,
        ],
    )
    def gather(x_hbm, row_hbm, col_hbm, gr_hbm, gc_hbm,
               idx_r, idx_c, buf_r, buf_c, sem_r, sem_c):
        c = lax.axis_index("c")
        s = lax.axis_index("s")
        wid = s * NC + c
        base = wid * epw
        pltpu.sync_copy(row_hbm.at[wid], idx_r)
        pltpu.sync_copy(col_hbm.at[wid], idx_c)

        def body(t, carry):
            off = base + t * CH
            cp_r = pltpu.async_copy(x_hbm.at[idx_r.at[t]], buf_r, sem_r)
            cp_c = pltpu.async_copy(x_hbm.at[idx_c.at[t]], buf_c, sem_c)
            cp_r.wait()
            pltpu.sync_copy(buf_r, gr_hbm.at[pl.ds(off, CH)])
            cp_c.wait()
            pltpu.sync_copy(buf_c, gc_hbm.at[pl.ds(off, CH)])
            return carry

        lax.fori_loop(0, cpw, body, 0)

    return gather


@functools.cache
def _scatter_fn(npad, d, e):
    nchk = e // CHS
    npr = npad // NW
    mesh = plsc.VectorSubcoreMesh(core_axis_name="c", subcore_axis_name="s")

    @functools.partial(
        pl.kernel, mesh=mesh,
        compiler_params=pltpu.CompilerParams(needs_layout_passes=False),
        out_type=[jax.ShapeDtypeStruct((npad, d), jnp.float32),
                  jax.ShapeDtypeStruct((npad, 16), jnp.float32)],
        scratch_types=[
            pltpu.VMEM((npr, d), jnp.float32),
            pltpu.VMEM((npr, 16), jnp.float32),
            pltpu.VMEM((1, CHS), jnp.int32),
            pltpu.VMEM((CHS + 16,), jnp.int32),
            pltpu.VMEM((16,), jnp.int32),
            pltpu.VMEM((16, d), jnp.float32),
            pltpu.SemaphoreType.DMA,
        ],
    )
    def scatter(msg_hbm, row2_hbm, zacc_hbm, zcnt_hbm, s_hbm, c_hbm,
                acc, cnt, ibuf, cid, eidb, gbuf, sem):
        c = lax.axis_index("c")
        s = lax.axis_index("s")
        wid = s * NC + c
        lo = wid * npr
        iota = lax.iota(jnp.int32, 16)
        zero16i = jnp.zeros((16,), jnp.int32)
        ones16 = jnp.full((16,), 1.0, jnp.float32)
        pltpu.sync_copy(zacc_hbm, acc)
        pltpu.sync_copy(zcnt_hbm, cnt)
        for g in range((CHS + 16) // 16):
            cid[pl.ds(g * 16, 16)] = zero16i

        def chunk_body(tt, carry):
            t2 = lax.rem(tt + wid * (nchk // NW), nchk)
            cb = t2 * CHS
            pltpu.sync_copy(row2_hbm.at[t2], ibuf)

            def scan_body(g, k):
                v = ibuf[0, pl.ds(g * 16, 16)]
                m = (v >= lo) & (v < lo + npr)
                mi = jnp.where(m, 1, 0)
                packed = jnp.where(m, v * 512 + (g * 16 + iota),
                                   jnp.full((16,), 0x7FFFFFFF, jnp.int32))
                pos = jnp.where(m, k + plsc.cumsum(mi) - 1,
                                jnp.full((16,), CHS, jnp.int32) + iota)
                plsc.store_scatter(cid, [pos], packed)
                return k + jnp.sum(mi)

            k = lax.fori_loop(0, CHS // 16, scan_body, 0)

            def blk_body(b, carry2):
                pk = cid[pl.ds(b * 16, 16)]
                eidb[...] = cb + lax.rem(jnp.abs(pk), 512)
                pltpu.async_copy(msg_hbm.at[eidb], gbuf, sem).wait()
                lim = k - b * 16
                for q in range(16):
                    @pl.when(q < lim)
                    def _(q=q):
                        lr = lax.shift_right_logical(_splat(pk, q), 9) - lo
                        plsc.addupdate_scatter(cnt, [lr, iota], ones16)
                        for j in range(d // 16):
                            plsc.addupdate_scatter(
                                acc, [lr, iota + j * 16],
                                gbuf[q, pl.ds(j * 16, 16)])
                return carry2

            lax.fori_loop(0, (k + 15) // 16, blk_body, 0)
            return carry

        lax.fori_loop(0, nchk, chunk_body, 0)
        pltpu.sync_copy(acc, s_hbm.at[pl.ds(lo, npr)])
        pltpu.sync_copy(cnt, c_hbm.at[pl.ds(lo, npr)])

    return scatter


def _edge_body(gr, gc, ea, w1r, w1c, w1a, b1, w2, b2,
               wpx, wpe, bp, wp2, bp2, nea_ref, msg_ref):
    dot = functools.partial(jnp.dot, preferred_element_type=jnp.float32)
    h1 = dot(gr[...], w1r[...]) + dot(gc[...], w1c[...]) + dot(ea[...], w1a[...])
    h1 = jax.nn.relu(h1 + b1[...])
    nea = dot(h1, w2[...]) + b2[...]
    nea_ref[...] = nea
    h2 = jax.nn.relu(dot(gc[...], wpx[...]) + dot(nea, wpe[...]) + bp[...])
    msg_ref[...] = dot(h2, wp2[...]) + bp2[...]


@functools.cache
def _edge_call(e, d, be):
    row_spec = pl.BlockSpec((be, d), lambda i: (i, 0))
    w_spec = pl.BlockSpec((d, d), lambda i: (0, 0))
    b_spec = pl.BlockSpec((1, d), lambda i: (0, 0))
    return pl.pallas_call(
        _edge_body,
        grid=(e // be,),
        in_specs=[row_spec, row_spec, row_spec,
                  w_spec, w_spec, w_spec, b_spec, w_spec, b_spec,
                  w_spec, w_spec, b_spec, w_spec, b_spec],
        out_specs=[row_spec, row_spec],
        out_shape=[jax.ShapeDtypeStruct((e, d), jnp.float32),
                   jax.ShapeDtypeStruct((e, d), jnp.float32)],
    )


def _node_body(x, s, cnt, wgx, wga, bg, wg2, bg2, wb, bb, wb2, bb2, out_ref):
    dot = functools.partial(jnp.dot, preferred_element_type=jnp.float32)
    agg = s[...] / jnp.maximum(cnt[:, 0:1], 1.0)
    h = jax.nn.relu(dot(x[...], wgx[...]) + dot(agg, wga[...]) + bg[...])
    xg = dot(h, wg2[...]) + bg2[...]
    xb = dot(jax.nn.relu(dot(xg, wb[...]) + bb[...]), wb2[...]) + bb2[...]
    out_ref[...] = xg + xb


@functools.cache
def _node_call(n, d, bn):
    row_spec = pl.BlockSpec((bn, d), lambda i: (i, 0))
    w_spec = pl.BlockSpec((d, d), lambda i: (0, 0))
    b_spec = pl.BlockSpec((1, d), lambda i: (0, 0))
    return pl.pallas_call(
        _node_body,
        grid=(n // bn,),
        in_specs=[row_spec,
                  row_spec,
                  pl.BlockSpec((bn, 16), lambda i: (i, 0)),
                  w_spec, w_spec, b_spec, w_spec, b_spec,
                  w_spec, b_spec, w_spec, b_spec],
        out_specs=pl.BlockSpec((bn, d), lambda i: (i, 0)),
        out_shape=jax.ShapeDtypeStruct((n, d), jnp.float32),
    )


def kernel(x, edge_index, edge_attr, We1, be1, We2, be2, Wp1, bp1, Wp2, bp2,
           Wg1, bg1, Wg2, bg2, Wb1, bb1, Wb2, bb2):
    n, d = x.shape
    e = edge_attr.shape[0]
    epw = e // NW
    cpw = epw // CH
    npad = -(-n // (NW * 8)) * (NW * 8)
    npr = npad // NW

    row = edge_index[0].astype(jnp.int32)
    col = edge_index[1].astype(jnp.int32)
    row3 = row.reshape(NW, cpw, CH)
    col3 = col.reshape(NW, cpw, CH)

    gr, gc = _gather_fn(n, d, e)(x, row3, col3)

    b2d = lambda b: b.reshape(1, d)
    nea, msg = _edge_call(e, d, 512)(
        gr, gc, edge_attr,
        We1[:d], We1[d:2 * d], We1[2 * d:], b2d(be1), We2, b2d(be2),
        Wp1[:d], Wp1[d:], b2d(bp1), Wp2, b2d(bp2))

    sums, counts = _scatter_fn(npad, d, e)(
        msg, row.reshape(e // CHS, 1, CHS),
        jnp.zeros((npr, d), jnp.float32),
        jnp.zeros((npr, 16), jnp.float32))

    out1 = _node_call(n, d, 1000)(
        x, sums[:n], counts[:n],
        Wg1[:d], Wg1[d:], b2d(bg1), Wg2, b2d(bg2),
        Wb1, b2d(bb1), Wb2, b2d(bb2))

    return out1, nea


# speedup vs baseline: 1.3781x; 1.2175x over previous
import functools

import jax
import jax.numpy as jnp
from jax import lax
from jax.experimental import pallas as pl
from jax.experimental.pallas import tpu as pltpu
from jax.experimental.pallas import tpu_sc as plsc

NC = 2
NS = 16
NW = NC * NS
CH = 80
CHS = 800
OFFB = 1024
SHB = 10

_GDN = lax.GatherDimensionNumbers(
    offset_dims=(), collapsed_slice_dims=(0,), start_index_map=(0,))


def _splat(vec, q):
    idx = jnp.full((16, 1), q, jnp.int32)
    return lax.gather(vec, idx, dimension_numbers=_GDN, slice_sizes=(1,),
                      mode=lax.GatherScatterMode.PROMISE_IN_BOUNDS)


@functools.cache
def _gather_fn(n, d, e):
    epw = e // NW
    cpw = epw // CH
    mesh = plsc.VectorSubcoreMesh(core_axis_name="c", subcore_axis_name="s")

    @functools.partial(
        pl.kernel, mesh=mesh,
        out_type=[jax.ShapeDtypeStruct((e, d), jnp.float32),
                  jax.ShapeDtypeStruct((e, d), jnp.float32)],
        scratch_types=[
            pltpu.VMEM((cpw, CH), jnp.int32),
            pltpu.VMEM((cpw, CH), jnp.int32),
            pltpu.VMEM((CH, d), jnp.float32),
            pltpu.VMEM((CH, d), jnp.float32),
            pltpu.VMEM((CH, d), jnp.float32),
            pltpu.VMEM((CH, d), jnp.float32),
            pltpu.SemaphoreType.DMA,
            pltpu.SemaphoreType.DMA,
            pltpu.SemaphoreType.DMA,
            pltpu.SemaphoreType.DMA,
        ],
    )
    def gather(x_hbm, row_hbm, col_hbm, gr_hbm, gc_hbm,
               idx_r, idx_c, bra, bca, brb, bcb, sra, sca, srb, scb):
        c = lax.axis_index("c")
        s = lax.axis_index("s")
        wid = s * NC + c
        base = wid * epw
        pltpu.sync_copy(row_hbm.at[wid], idx_r)
        pltpu.sync_copy(col_hbm.at[wid], idx_c)

        def issue(t, br, bc, sr, sc):
            pltpu.async_copy(x_hbm.at[idx_r.at[t]], br, sr)
            pltpu.async_copy(x_hbm.at[idx_c.at[t]], bc, sc)

        def drain_write(t, br, bc, sr, sc):
            off = base + t * CH
            pltpu.make_async_copy(x_hbm.at[idx_r.at[t]], br, sr).wait()
            pltpu.sync_copy(br, gr_hbm.at[pl.ds(off, CH)])
            pltpu.make_async_copy(x_hbm.at[idx_c.at[t]], bc, sc).wait()
            pltpu.sync_copy(bc, gc_hbm.at[pl.ds(off, CH)])

        issue(0, bra, bca, sra, sca)

        def body(p, carry):
            t0 = p * 2
            issue(t0 + 1, brb, bcb, srb, scb)
            drain_write(t0, bra, bca, sra, sca)
            issue(t0 + 2, bra, bca, sra, sca)
            drain_write(t0 + 1, brb, bcb, srb, scb)
            return carry

        lax.fori_loop(0, cpw // 2, body, 0)
        drain_write(cpw - 1, bra, bca, sra, sca)

    return gather


@functools.cache
def _scatter_fn(npad, d, e):
    nchk = e // CHS
    npr = npad // NW
    mesh = plsc.VectorSubcoreMesh(core_axis_name="c", subcore_axis_name="s")

    @functools.partial(
        pl.kernel, mesh=mesh,
        compiler_params=pltpu.CompilerParams(needs_layout_passes=False),
        out_type=[jax.ShapeDtypeStruct((npad, d), jnp.float32),
                  jax.ShapeDtypeStruct((npad, 16), jnp.float32)],
        scratch_types=[
            pltpu.VMEM((npr, d), jnp.float32),
            pltpu.VMEM((npr, 16), jnp.float32),
            pltpu.VMEM((1, CHS), jnp.int32),
            pltpu.VMEM((1, CHS), jnp.int32),
            pltpu.VMEM((CHS + 16,), jnp.int32),
            pltpu.VMEM((16,), jnp.int32),
            pltpu.VMEM((16,), jnp.int32),
            pltpu.VMEM((16, d), jnp.float32),
            pltpu.VMEM((16, d), jnp.float32),
            pltpu.SemaphoreType.DMA,
            pltpu.SemaphoreType.DMA,
            pltpu.SemaphoreType.DMA,
            pltpu.SemaphoreType.DMA,
        ],
    )
    def scatter(msg_hbm, row2_hbm, zacc_hbm, zcnt_hbm, s_hbm, c_hbm,
                acc, cnt, iba, ibb, cid, eida, eidb, gba, gbb,
                semi, semj, sema, semb):
        c = lax.axis_index("c")
        s = lax.axis_index("s")
        wid = s * NC + c
        lo = wid * npr
        iota = lax.iota(jnp.int32, 16)
        zero16i = jnp.zeros((16,), jnp.int32)
        ones16 = jnp.full((16,), 1.0, jnp.float32)
        pltpu.sync_copy(zacc_hbm, acc)
        pltpu.sync_copy(zcnt_hbm, cnt)
        for g in range((CHS + 16) // 16):
            cid[pl.ds(g * 16, 16)] = zero16i
        start = wid * (nchk // NW)

        def t2_of(tt):
            return lax.rem(tt + start, nchk)

        def process_chunk(t2, ib):
            cb = t2 * CHS

            def scan_body(g, k):
                v = ib[0, pl.ds(g * 16, 16)]
                m = (v >= lo) & (v < lo + npr)
                mi = jnp.where(m, 1, 0)
                kk = jnp.sum(mi)

                @pl.when(kk > 0)
                def _():
                    packed = jnp.where(m, v * OFFB + (g * 16 + iota),
                                       jnp.full((16,), 0x7FFF0000, jnp.int32))
                    pos = jnp.where(m, k + plsc.cumsum(mi) - 1,
                                    jnp.full((16,), CHS, jnp.int32) + iota)
                    plsc.store_scatter(cid, [pos], packed)

                return k + kk

            k = lax.fori_loop(0, CHS // 16, scan_body, 0)
            nb = (k + 15) // 16

            def eid_of(b):
                pk = cid[pl.ds(b * 16, 16)]
                return cb + lax.rem(jnp.abs(pk), OFFB)

            def process_blk(b, gb):
                pk = cid[pl.ds(b * 16, 16)]
                lim = k - b * 16
                for q in range(16):
                    @pl.when(q < lim)
                    def _(q=q):
                        lr = lax.shift_right_logical(_splat(pk, q), SHB) - lo
                        plsc.addupdate_scatter(cnt, [lr, iota], ones16)
                        for j in range(d // 16):
                            plsc.addupdate_scatter(
                                acc, [lr, iota + j * 16],
                                gb[q, pl.ds(j * 16, 16)])

            @pl.when(nb > 0)
            def _():
                eida[...] = eid_of(0)
                pltpu.async_copy(msg_hbm.at[eida], gba, sema)

            def blk_pair(p2, carry2):
                b0 = p2 * 2
                b1 = b0 + 1

                @pl.when(b1 < nb)
                def _():
                    eidb[...] = eid_of(b1)
                    pltpu.async_copy(msg_hbm.at[eidb], gbb, semb)

                pltpu.make_async_copy(msg_hbm.at[eida], gba, sema).wait()
                process_blk(b0, gba)

                @pl.when(b0 + 2 < nb)
                def _():
                    eida[...] = eid_of(b0 + 2)
                    pltpu.async_copy(msg_hbm.at[eida], gba, sema)

                @pl.when(b1 < nb)
                def _():
                    pltpu.make_async_copy(msg_hbm.at[eidb], gbb, semb).wait()
                    process_blk(b1, gbb)

                return carry2

            lax.fori_loop(0, (nb + 1) // 2, blk_pair, 0)

        pltpu.async_copy(row2_hbm.at[t2_of(0)], iba, semi)

        def chunk_pair(p, carry):
            tt0 = p * 2
            pltpu.async_copy(row2_hbm.at[t2_of(tt0 + 1)], ibb, semj)
            pltpu.make_async_copy(row2_hbm.at[t2_of(tt0)], iba, semi).wait()
            process_chunk(t2_of(tt0), iba)

            @pl.when(tt0 + 2 < nchk)
            def _():
                pltpu.async_copy(row2_hbm.at[t2_of(tt0 + 2)], iba, semi)

            pltpu.make_async_copy(row2_hbm.at[t2_of(tt0 + 1)], ibb, semj).wait()
            process_chunk(t2_of(tt0 + 1), ibb)
            return carry

        lax.fori_loop(0, nchk // 2, chunk_pair, 0)
        pltpu.sync_copy(acc, s_hbm.at[pl.ds(lo, npr)])
        pltpu.sync_copy(cnt, c_hbm.at[pl.ds(lo, npr)])

    return scatter


def _edge_body(gr, gc, ea, w1r, w1c, w1a, b1, w2, b2,
               wpx, wpe, bp, wp2, bp2, nea_ref, msg_ref):
    bf = lambda a: a.astype(jnp.bfloat16)

    def dot(a, w):
        return jnp.dot(bf(a), bf(w), preferred_element_type=jnp.float32)

    h1 = dot(gr[...], w1r[...]) + dot(gc[...], w1c[...]) + dot(ea[...], w1a[...])
    h1 = jax.nn.relu(h1 + b1[...])
    nea = dot(h1, w2[...]) + b2[...]
    nea_ref[...] = nea
    h2 = jax.nn.relu(dot(gc[...], wpx[...]) + dot(nea, wpe[...]) + bp[...])
    msg_ref[...] = dot(h2, wp2[...]) + bp2[...]


@functools.cache
def _edge_call(e, d, be):
    row_spec = pl.BlockSpec((be, d), lambda i: (i, 0))
    w_spec = pl.BlockSpec((d, d), lambda i: (0, 0))
    b_spec = pl.BlockSpec((1, d), lambda i: (0, 0))
    return pl.pallas_call(
        _edge_body,
        grid=(e // be,),
        in_specs=[row_spec, row_spec, row_spec,
                  w_spec, w_spec, w_spec, b_spec, w_spec, b_spec,
                  w_spec, w_spec, b_spec, w_spec, b_spec],
        out_specs=[row_spec, row_spec],
        out_shape=[jax.ShapeDtypeStruct((e, d), jnp.float32),
                   jax.ShapeDtypeStruct((e, d), jnp.float32)],
    )


def _node_body(x, s, cnt, wgx, wga, bg, wg2, bg2, wb, bb, wb2, bb2, out_ref):
    dot = functools.partial(jnp.dot, preferred_element_type=jnp.float32)
    agg = s[...] / jnp.maximum(cnt[:, 0:1], 1.0)
    h = jax.nn.relu(dot(x[...], wgx[...]) + dot(agg, wga[...]) + bg[...])
    xg = dot(h, wg2[...]) + bg2[...]
    xb = dot(jax.nn.relu(dot(xg, wb[...]) + bb[...]), wb2[...]) + bb2[...]
    out_ref[...] = xg + xb


@functools.cache
def _node_call(n, d, bn):
    row_spec = pl.BlockSpec((bn, d), lambda i: (i, 0))
    w_spec = pl.BlockSpec((d, d), lambda i: (0, 0))
    b_spec = pl.BlockSpec((1, d), lambda i: (0, 0))
    return pl.pallas_call(
        _node_body,
        grid=(n // bn,),
        in_specs=[row_spec,
                  row_spec,
                  pl.BlockSpec((bn, 16), lambda i: (i, 0)),
                  w_spec, w_spec, b_spec, w_spec, b_spec,
                  w_spec, b_spec, w_spec, b_spec],
        out_specs=pl.BlockSpec((bn, d), lambda i: (i, 0)),
        out_shape=jax.ShapeDtypeStruct((n, d), jnp.float32),
    )


def kernel(x, edge_index, edge_attr, We1, be1, We2, be2, Wp1, bp1, Wp2, bp2,
           Wg1, bg1, Wg2, bg2, Wb1, bb1, Wb2, bb2):
    n, d = x.shape
    e = edge_attr.shape[0]
    epw = e // NW
    cpw = epw // CH
    npad = -(-n // (NW * 8)) * (NW * 8)
    npr = npad // NW

    row = edge_index[0].astype(jnp.int32)
    col = edge_index[1].astype(jnp.int32)
    row3 = row.reshape(NW, cpw, CH)
    col3 = col.reshape(NW, cpw, CH)

    gr, gc = _gather_fn(n, d, e)(x, row3, col3)

    b2d = lambda b: b.reshape(1, d)
    nea, msg = _edge_call(e, d, 512)(
        gr, gc, edge_attr,
        We1[:d], We1[d:2 * d], We1[2 * d:], b2d(be1), We2, b2d(be2),
        Wp1[:d], Wp1[d:], b2d(bp1), Wp2, b2d(bp2))

    sums, counts = _scatter_fn(npad, d, e)(
        msg, row.reshape(e // CHS, 1, CHS),
        jnp.zeros((npr, d), jnp.float32),
        jnp.zeros((npr, 16), jnp.float32))

    out1 = _node_call(n, d, 1000)(
        x, sums[:n], counts[:n],
        Wg1[:d], Wg1[d:], b2d(bg1), Wg2, b2d(bg2),
        Wb1, b2d(bb1), Wb2, b2d(bb2))

    return out1, nea
